# Initial kernel scaffold; baseline (speedup 1.0000x reference)
import jax
import jax.numpy as jnp
from jax.experimental import pallas as pl


def kernel(feature, indices, feature_volume, count_volume):
    raise NotImplementedError("write your pallas kernel here")


# speedup vs baseline: 2.9864x; 2.9864x over previous
import jax
import jax.numpy as jnp
from jax.experimental import pallas as pl
from jax.experimental.pallas import tpu as pltpu


def _combine_body(f_ref, w_ref, fv_ref, cv_ref, of_ref, oc_ref):
    f = f_ref[...]
    w = w_ref[...]
    fv = fv_ref[...]
    cv = cv_ref[...]
    touched = w > 0.0
    pooled = f / w
    cv1 = cv + 1.0
    of_ref[...] = jnp.where(touched, fv * cv + pooled / cv1, fv)
    oc_ref[...] = jnp.where(touched, cv1, cv)


def _combine(fcache, wcache, fv, cv):
    n = 65536
    blk = 2048
    grid = n // blk
    spec = pl.BlockSpec((blk, 256), lambda i: (i, 0))
    return pl.pallas_call(
        _combine_body,
        grid=(grid,),
        in_specs=[spec, spec, spec, spec],
        out_specs=[spec, spec],
        out_shape=[
            jax.ShapeDtypeStruct((n, 256), jnp.float32),
            jax.ShapeDtypeStruct((n, 256), jnp.float32),
        ],
    )(fcache, wcache, fv, cv)


def kernel(feature, indices, feature_volume, count_volume):
    xs, ys, zs = feature_volume.shape
    npts = feature.size
    feat = feature.reshape(npts)
    idx = indices.reshape(npts, 3)
    flat = ys * zs * idx[:, 0] + zs * idx[:, 1] + idx[:, 2]
    fcache = jnp.zeros(xs * ys * zs, jnp.float32).at[flat].add(feat, mode="drop")
    wcache = jnp.zeros(xs * ys * zs, jnp.float32).at[flat].add(1.0, mode="drop")
    of, oc = _combine(
        fcache.reshape(65536, 256),
        wcache.reshape(65536, 256),
        feature_volume.reshape(65536, 256),
        count_volume.reshape(65536, 256),
    )
    return of.reshape(xs, ys, zs), oc.reshape(xs, ys, zs)


# speedup vs baseline: 8.6657x; 2.9017x over previous
import functools

import jax
import jax.numpy as jnp
from jax import lax
from jax.experimental import pallas as pl
from jax.experimental.pallas import tpu as pltpu
from jax.experimental.pallas import tpu_sc as plsc

NPTS = 8 * 512 * 512
VOL = 256 * 256 * 256
NSUB = 16
NCORE = 2
TPP = NPTS // NSUB
CH = 2048
NCHUNK = TPP // CH
SLICE = 256 * 256
XS_PER_ROUND = 13
HALF_X = 128
ROUND_WIDTHS = [XS_PER_ROUND] * (HALF_X // XS_PER_ROUND) + (
    [HALF_X % XS_PER_ROUND] if HALF_X % XS_PER_ROUND else [])
NV_MAX = XS_PER_ROUND * SLICE
DUMP_SZ = 4096
ACC = NV_MAX + DUMP_SZ
ZB = 8192


def _sc_body(x_hbm, y_hbm, z_hbm, feat_hbm, f_out, w_out,
             xb, yb, zb_, fb, ib, ones, zeros, facc, wacc):
    c = lax.axis_index("c")
    s = lax.axis_index("s")
    tile_base = s * TPP

    def init(i, _):
        zeros[pl.ds(i * 16, 16)] = jnp.zeros((16,), jnp.float32)
        @pl.when(i < CH // 16)
        def _():
            ones[pl.ds(i * 16, 16)] = jnp.ones((16,), jnp.float32)
        return 0
    lax.fori_loop(0, ZB // 16, init, 0)

    for r, width in enumerate(ROUND_WIDTHS):
        nv = width * SLICE
        base = (c * HALF_X + r * XS_PER_ROUND) * SLICE
        per_tile = nv // NSUB

        full, rem = per_tile // ZB, per_tile % ZB
        for j in range(full):
            off = s * per_tile + j * ZB
            pltpu.sync_copy(zeros, facc.at[pl.ds(off, ZB)])
            pltpu.sync_copy(zeros, wacc.at[pl.ds(off, ZB)])
        if rem:
            off = s * per_tile + full * ZB
            pltpu.sync_copy(zeros.at[pl.ds(0, rem)], facc.at[pl.ds(off, rem)])
            pltpu.sync_copy(zeros.at[pl.ds(0, rem)], wacc.at[pl.ds(off, rem)])
        plsc.subcore_barrier()

        def chunk(k, _):
            p0 = tile_base + k * CH
            pltpu.sync_copy(x_hbm.at[pl.ds(p0, CH)], xb)
            pltpu.sync_copy(y_hbm.at[pl.ds(p0, CH)], yb)
            pltpu.sync_copy(z_hbm.at[pl.ds(p0, CH)], zb_)
            pltpu.sync_copy(feat_hbm.at[pl.ds(p0, CH)], fb)

            def vec(i, _):
                sl = pl.ds(i * 16, 16)
                flat = (xb[sl] << 16) | (yb[sl] << 8) | zb_[sl]
                t = flat - base
                m = (t >= 0) & (t < nv)
                ib[sl] = jnp.where(m, t, NV_MAX + (flat & (DUMP_SZ - 1)))
                return 0
            lax.fori_loop(0, CH // 16, vec, 0)

            pltpu.sync_copy(fb, facc.at[ib], add=True)
            pltpu.sync_copy(ones, wacc.at[ib], add=True)
            return 0
        lax.fori_loop(0, NCHUNK, chunk, 0)
        plsc.subcore_barrier()

        dr = nv // NSUB
        hbase = base + s * dr
        pltpu.sync_copy(facc.at[pl.ds(s * dr, dr)], f_out.at[pl.ds(hbase, dr)])
        pltpu.sync_copy(wacc.at[pl.ds(s * dr, dr)], w_out.at[pl.ds(hbase, dr)])
        plsc.subcore_barrier()


@functools.partial(jax.jit, static_argnums=())
def _sc_scatter(x, y, z, feat):
    mesh = plsc.VectorSubcoreMesh(core_axis_name="c", subcore_axis_name="s")
    return pl.kernel(
        _sc_body,
        out_type=[
            jax.ShapeDtypeStruct((VOL,), jnp.float32),
            jax.ShapeDtypeStruct((VOL,), jnp.float32),
        ],
        mesh=mesh,
        scratch_types=[
            pltpu.VMEM((CH,), jnp.int32),
            pltpu.VMEM((CH,), jnp.int32),
            pltpu.VMEM((CH,), jnp.int32),
            pltpu.VMEM((CH,), jnp.float32),
            pltpu.VMEM((CH,), jnp.int32),
            pltpu.VMEM((CH,), jnp.float32),
            pltpu.VMEM((ZB,), jnp.float32),
            pltpu.VMEM_SHARED((ACC,), jnp.float32),
            pltpu.VMEM_SHARED((ACC,), jnp.float32),
        ],
    )(x, y, z, feat)


def _combine_body(f_ref, w_ref, fv_ref, cv_ref, of_ref, oc_ref):
    f = f_ref[...]
    w = w_ref[...]
    fv = fv_ref[...]
    cv = cv_ref[...]
    touched = w > 0.0
    pooled = f / w
    cv1 = cv + 1.0
    of_ref[...] = jnp.where(touched, fv * cv + pooled / cv1, fv)
    oc_ref[...] = jnp.where(touched, cv1, cv)


def _combine(fcache, wcache, fv, cv):
    n = 65536
    blk = 2048
    spec = pl.BlockSpec((blk, 256), lambda i: (i, 0))
    return pl.pallas_call(
        _combine_body,
        grid=(n // blk,),
        in_specs=[spec, spec, spec, spec],
        out_specs=[spec, spec],
        out_shape=[
            jax.ShapeDtypeStruct((n, 256), jnp.float32),
            jax.ShapeDtypeStruct((n, 256), jnp.float32),
        ],
    )(fcache, wcache, fv, cv)


def kernel(feature, indices, feature_volume, count_volume):
    xs, ys, zs = feature_volume.shape
    feat = feature.reshape(NPTS)
    idx = indices.reshape(NPTS, 3)
    x = idx[:, 0]
    y = idx[:, 1]
    z = idx[:, 2]
    fcache, wcache = _sc_scatter(x, y, z, feat)
    of, oc = _combine(
        fcache.reshape(65536, 256),
        wcache.reshape(65536, 256),
        feature_volume.reshape(65536, 256),
        count_volume.reshape(65536, 256),
    )
    return of.reshape(xs, ys, zs), oc.reshape(xs, ys, zs)


# speedup vs baseline: 10.4834x; 1.2098x over previous
import functools

import jax
import jax.numpy as jnp
from jax import lax
from jax.experimental import pallas as pl
from jax.experimental.pallas import tpu as pltpu
from jax.experimental.pallas import tpu_sc as plsc

NPTS = 8 * 512 * 512
VOL = 256 * 256 * 256
NSUB = 16
TPP = NPTS // NSUB
CH = 2048
NCHUNK = TPP // CH
SLICE = 256 * 256
XS_PER_ROUND = 10
HALF_X = 128
ROUND_WIDTHS = [XS_PER_ROUND] * (HALF_X // XS_PER_ROUND) + (
    [HALF_X % XS_PER_ROUND] if HALF_X % XS_PER_ROUND else [])
NV_MAX = XS_PER_ROUND * SLICE
DUMP_SZ = 4096
ACC = NV_MAX + DUMP_SZ
ZB = 8192
NROW = 64
RW = 128
RING = NROW * RW


def _sc_body(x_hbm, y_hbm, z_hbm, feat_hbm, f_out, w_out,
             xb0, xb1, yb0, yb1, zb0, zb1, fb0, fb1,
             ones_row, zeros, wcur, irows, frows, facc, wacc, sem0, sem1):
    c = lax.axis_index("c")
    s = lax.axis_index("s")
    tile_base = s * TPP
    xb = [xb0, xb1]
    yb = [yb0, yb1]
    zb = [zb0, zb1]
    fb = [fb0, fb1]
    sems = [sem0, sem1]
    iota = lax.iota(jnp.int32, 16)

    def init(i, _):
        zeros[pl.ds(i * 16, 16)] = jnp.zeros((16,), jnp.float32)
        @pl.when(i < RW // 16)
        def _():
            ones_row[pl.ds(i * 16, 16)] = jnp.ones((16,), jnp.float32)
        return 0
    lax.fori_loop(0, ZB // 16, init, 0)
    wcur[pl.ds(0, 16)] = jnp.zeros((16,), jnp.int32)

    def issue(k, b):
        p0 = tile_base + k * CH
        pltpu.async_copy(x_hbm.at[pl.ds(p0, CH)], xb[b], sems[b])
        pltpu.async_copy(y_hbm.at[pl.ds(p0, CH)], yb[b], sems[b])
        pltpu.async_copy(z_hbm.at[pl.ds(p0, CH)], zb[b], sems[b])
        pltpu.async_copy(feat_hbm.at[pl.ds(p0, CH)], fb[b], sems[b])

    def drain(k, b):
        p0 = tile_base + k * CH
        pltpu.make_async_copy(x_hbm.at[pl.ds(p0, CH)], xb[b], sems[b]).wait()
        pltpu.make_async_copy(y_hbm.at[pl.ds(p0, CH)], yb[b], sems[b]).wait()
        pltpu.make_async_copy(z_hbm.at[pl.ds(p0, CH)], zb[b], sems[b]).wait()
        pltpu.make_async_copy(feat_hbm.at[pl.ds(p0, CH)], fb[b], sems[b]).wait()

    def flush_rows(w_scalar, f_scalar):
        nfull = (w_scalar >> 7) - f_scalar

        def fl(j, _):
            j2 = (f_scalar + j) & (NROW - 1)
            pltpu.sync_copy(frows.at[j2], facc.at[irows.at[j2]], add=True)
            pltpu.sync_copy(ones_row, wacc.at[irows.at[j2]], add=True)
            return 0
        lax.fori_loop(0, nfull, fl, 0)
        return f_scalar + nfull


    fcur = jnp.int32(0)
    for r, width in enumerate(ROUND_WIDTHS):
        nv = width * SLICE
        base = (c * HALF_X + r * XS_PER_ROUND) * SLICE
        per_tile = nv // NSUB

        full, rem = per_tile // ZB, per_tile % ZB
        for j in range(full):
            off = s * per_tile + j * ZB
            pltpu.sync_copy(zeros, facc.at[pl.ds(off, ZB)])
            pltpu.sync_copy(zeros, wacc.at[pl.ds(off, ZB)])
        if rem:
            off = s * per_tile + full * ZB
            pltpu.sync_copy(zeros.at[pl.ds(0, rem)], facc.at[pl.ds(off, rem)])
            pltpu.sync_copy(zeros.at[pl.ds(0, rem)], wacc.at[pl.ds(off, rem)])
        plsc.subcore_barrier()

        issue(0, 0)

        def proc(k, bb, f_):
            drain(k, bb)

            @pl.when(k + 1 < NCHUNK)
            def _():
                issue(k + 1, 1 - bb)

            def vec(i, _):
                sl = pl.ds(i * 16, 16)
                wvi = wcur[pl.ds(0, 16)]
                flat = (xb[bb][sl] << 16) | (yb[bb][sl] << 8) | zb[bb][sl]
                t = flat - base
                m = (t >= 0) & (t < nv)
                dest = (wvi + jnp.cumsum(jnp.where(m, 1, 0)) - 1) & (RING - 1)
                plsc.store_scatter(irows, [dest >> 7, dest & (RW - 1)], t,
                                   mask=m)
                plsc.store_scatter(frows, [dest >> 7, dest & (RW - 1)],
                                   fb[bb][sl], mask=m)
                wcur[pl.ds(0, 16)] = wvi + plsc.all_reduce_population_count(m)
                return 0
            lax.fori_loop(0, CH // 16, vec, 0)
            return flush_rows(jnp.max(wcur[pl.ds(0, 16)]), f_)

        def chunk2(i, f_):
            f_ = proc(2 * i, 0, f_)
            f_ = proc(2 * i + 1, 1, f_)
            return f_
        fcur = lax.fori_loop(0, NCHUNK // 2, chunk2, fcur)

        w_s = jnp.max(wcur[pl.ds(0, 16)])
        for k2 in range(8):
            p = (w_s + 16 * k2 + iota) & (RING - 1)
            plsc.store_scatter(irows, [p >> 7, p & (RW - 1)],
                               NV_MAX + (p & (RW - 1)))
        w_pad = (w_s & ~(RW - 1)) + RW
        fcur = flush_rows(w_pad, fcur)
        wcur[pl.ds(0, 16)] = (wcur[pl.ds(0, 16)] * 0) + w_pad
        plsc.subcore_barrier()

        dr = nv // NSUB
        hbase = base + s * dr
        pltpu.sync_copy(facc.at[pl.ds(s * dr, dr)], f_out.at[pl.ds(hbase, dr)])
        pltpu.sync_copy(wacc.at[pl.ds(s * dr, dr)], w_out.at[pl.ds(hbase, dr)])
        plsc.subcore_barrier()


def _sc_scatter(x, y, z, feat):
    mesh = plsc.VectorSubcoreMesh(core_axis_name="c", subcore_axis_name="s")
    return pl.kernel(
        _sc_body,
        out_type=[
            jax.ShapeDtypeStruct((VOL,), jnp.float32),
            jax.ShapeDtypeStruct((VOL,), jnp.float32),
        ],
        mesh=mesh,
        compiler_params=pltpu.CompilerParams(needs_layout_passes=False),
        scratch_types=[
            pltpu.VMEM((CH,), jnp.int32),
            pltpu.VMEM((CH,), jnp.int32),
            pltpu.VMEM((CH,), jnp.int32),
            pltpu.VMEM((CH,), jnp.int32),
            pltpu.VMEM((CH,), jnp.int32),
            pltpu.VMEM((CH,), jnp.int32),
            pltpu.VMEM((CH,), jnp.float32),
            pltpu.VMEM((CH,), jnp.float32),
            pltpu.VMEM((RW,), jnp.float32),
            pltpu.VMEM((ZB,), jnp.float32),
            pltpu.VMEM((16,), jnp.int32),
            pltpu.VMEM((NROW, RW), jnp.int32),
            pltpu.VMEM((NROW, RW), jnp.float32),
            pltpu.VMEM_SHARED((ACC,), jnp.float32),
            pltpu.VMEM_SHARED((ACC,), jnp.float32),
            pltpu.SemaphoreType.DMA,
            pltpu.SemaphoreType.DMA,
        ],
    )(x, y, z, feat)


def _combine_body(f_ref, w_ref, fv_ref, cv_ref, of_ref, oc_ref):
    f = f_ref[...]
    w = w_ref[...]
    fv = fv_ref[...]
    cv = cv_ref[...]
    touched = w > 0.0
    pooled = f / w
    cv1 = cv + 1.0
    of_ref[...] = jnp.where(touched, fv * cv + pooled / cv1, fv)
    oc_ref[...] = jnp.where(touched, cv1, cv)


def _combine(fcache, wcache, fv, cv):
    n = 65536
    blk = 2048
    spec = pl.BlockSpec((blk, 256), lambda i: (i, 0))
    return pl.pallas_call(
        _combine_body,
        grid=(n // blk,),
        in_specs=[spec, spec, spec, spec],
        out_specs=[spec, spec],
        out_shape=[
            jax.ShapeDtypeStruct((n, 256), jnp.float32),
            jax.ShapeDtypeStruct((n, 256), jnp.float32),
        ],
    )(fcache, wcache, fv, cv)


def kernel(feature, indices, feature_volume, count_volume):
    xs, ys, zs = feature_volume.shape
    feat = feature.reshape(NPTS)
    idx = indices.reshape(NPTS, 3)
    x = idx[:, 0]
    y = idx[:, 1]
    z = idx[:, 2]
    fcache, wcache = _sc_scatter(x, y, z, feat)
    of, oc = _combine(
        fcache.reshape(65536, 256),
        wcache.reshape(65536, 256),
        feature_volume.reshape(65536, 256),
        count_volume.reshape(65536, 256),
    )
    return of.reshape(xs, ys, zs), oc.reshape(xs, ys, zs)


# speedup vs baseline: 21.2247x; 2.0246x over previous
import functools

import jax
import jax.numpy as jnp
from jax import lax
from jax.experimental import pallas as pl
from jax.experimental.pallas import tpu as pltpu
from jax.experimental.pallas import tpu_sc as plsc

NPTS = 8 * 512 * 512
VOL = 256 * 256 * 256
NSUB = 16
TPP = NPTS // NSUB
CH = 2048
NCHUNK = TPP // CH
SLICE = 256 * 256
XS_PER_ROUND = 10
HALF_X = 128
ROUND_WIDTHS = [XS_PER_ROUND] * (HALF_X // XS_PER_ROUND) + (
    [HALF_X % XS_PER_ROUND] if HALF_X % XS_PER_ROUND else [])
NV_MAX = XS_PER_ROUND * SLICE
DUMP_SZ = 4096
ACC = NV_MAX + DUMP_SZ
ZB = 8192
NROW = 64
RW = 128
RING = NROW * RW


def _sc_body(x_hbm, y_hbm, z_hbm, feat_hbm, f_out, w_out,
             xb0, xb1, yb0, yb1, zb0, zb1, fb0, fb1,
             ones_row, zeros, wcur, irows, frows, facc, wacc, sem0, sem1):
    c = lax.axis_index("c")
    s = lax.axis_index("s")
    tile_base = s * TPP
    xb = [xb0, xb1]
    yb = [yb0, yb1]
    zb = [zb0, zb1]
    fb = [fb0, fb1]
    sems = [sem0, sem1]
    iota = lax.iota(jnp.int32, 16)

    def init(i, _):
        zeros[pl.ds(i * 16, 16)] = jnp.zeros((16,), jnp.float32)
        @pl.when(i < RW // 16)
        def _():
            ones_row[pl.ds(i * 16, 16)] = jnp.ones((16,), jnp.float32)
        return 0
    lax.fori_loop(0, ZB // 16, init, 0)
    wcur[pl.ds(0, 16)] = jnp.zeros((16,), jnp.int32)

    def issue(k, b):
        p0 = tile_base + k * CH
        pltpu.async_copy(x_hbm.at[pl.ds(p0, CH)], xb[b], sems[b])
        pltpu.async_copy(y_hbm.at[pl.ds(p0, CH)], yb[b], sems[b])
        pltpu.async_copy(z_hbm.at[pl.ds(p0, CH)], zb[b], sems[b])
        pltpu.async_copy(feat_hbm.at[pl.ds(p0, CH)], fb[b], sems[b])

    def drain(k, b):
        p0 = tile_base + k * CH
        pltpu.make_async_copy(x_hbm.at[pl.ds(p0, CH)], xb[b], sems[b]).wait()
        pltpu.make_async_copy(y_hbm.at[pl.ds(p0, CH)], yb[b], sems[b]).wait()
        pltpu.make_async_copy(z_hbm.at[pl.ds(p0, CH)], zb[b], sems[b]).wait()
        pltpu.make_async_copy(feat_hbm.at[pl.ds(p0, CH)], fb[b], sems[b]).wait()

    def flush_rows(w_scalar, f_scalar):
        nfull = (w_scalar >> 7) - f_scalar

        def fl(j, _):
            j2 = (f_scalar + j) & (NROW - 1)
            pltpu.sync_copy(frows.at[j2], facc.at[irows.at[j2]], add=True)
            pltpu.sync_copy(ones_row, wacc.at[irows.at[j2]], add=True)
            return 0
        lax.fori_loop(0, nfull, fl, 0)
        return f_scalar + nfull


    fcur = jnp.int32(0)
    for r, width in enumerate(ROUND_WIDTHS):
        nv = width * SLICE
        base = (c * HALF_X + r * XS_PER_ROUND) * SLICE
        per_tile = nv // NSUB

        full, rem = per_tile // ZB, per_tile % ZB
        for j in range(full):
            off = s * per_tile + j * ZB
            pltpu.sync_copy(zeros, facc.at[pl.ds(off, ZB)])
            pltpu.sync_copy(zeros, wacc.at[pl.ds(off, ZB)])
        if rem:
            off = s * per_tile + full * ZB
            pltpu.sync_copy(zeros.at[pl.ds(0, rem)], facc.at[pl.ds(off, rem)])
            pltpu.sync_copy(zeros.at[pl.ds(0, rem)], wacc.at[pl.ds(off, rem)])
        plsc.subcore_barrier()

        issue(0, 0)

        def proc(k, bb, f_):
            drain(k, bb)

            @pl.when(k + 1 < NCHUNK)
            def _():
                issue(k + 1, 1 - bb)

            def group(g, _):
                ms, ts, fs, cs = [], [], [], []
                for j in range(8):
                    sl = pl.ds((g * 8 + j) * 16, 16)
                    flat = (xb[bb][sl] << 16) | (yb[bb][sl] << 8) | zb[bb][sl]
                    t = flat - base
                    m = (t >= 0) & (t < nv)
                    ms.append(m)
                    ts.append(t)
                    fs.append(fb[bb][sl])
                    cs.append(plsc.all_reduce_population_count(m))
                b = wcur[pl.ds(0, 16)]
                bases = []
                for j in range(8):
                    bases.append(b)
                    b = b + cs[j]
                wcur[pl.ds(0, 16)] = b
                for j in range(8):
                    dest = (bases[j] + jnp.cumsum(jnp.where(ms[j], 1, 0))
                            - 1) & (RING - 1)
                    plsc.store_scatter(irows, [dest >> 7, dest & (RW - 1)],
                                       ts[j], mask=ms[j])
                    plsc.store_scatter(frows, [dest >> 7, dest & (RW - 1)],
                                       fs[j], mask=ms[j])
                return 0
            lax.fori_loop(0, CH // 128, group, 0)
            return flush_rows(jnp.max(wcur[pl.ds(0, 16)]), f_)

        def chunk2(i, f_):
            f_ = proc(2 * i, 0, f_)
            f_ = proc(2 * i + 1, 1, f_)
            return f_
        fcur = lax.fori_loop(0, NCHUNK // 2, chunk2, fcur)

        w_s = jnp.max(wcur[pl.ds(0, 16)])
        for k2 in range(8):
            p = (w_s + 16 * k2 + iota) & (RING - 1)
            plsc.store_scatter(irows, [p >> 7, p & (RW - 1)],
                               NV_MAX + (p & (RW - 1)))
        w_pad = (w_s & ~(RW - 1)) + RW
        fcur = flush_rows(w_pad, fcur)
        wcur[pl.ds(0, 16)] = (wcur[pl.ds(0, 16)] * 0) + w_pad
        plsc.subcore_barrier()

        dr = nv // NSUB
        hbase = base + s * dr
        pltpu.sync_copy(facc.at[pl.ds(s * dr, dr)], f_out.at[pl.ds(hbase, dr)])
        pltpu.sync_copy(wacc.at[pl.ds(s * dr, dr)], w_out.at[pl.ds(hbase, dr)])
        plsc.subcore_barrier()


def _sc_scatter(x, y, z, feat):
    mesh = plsc.VectorSubcoreMesh(core_axis_name="c", subcore_axis_name="s")
    return pl.kernel(
        _sc_body,
        out_type=[
            jax.ShapeDtypeStruct((VOL,), jnp.float32),
            jax.ShapeDtypeStruct((VOL,), jnp.float32),
        ],
        mesh=mesh,
        compiler_params=pltpu.CompilerParams(needs_layout_passes=False),
        scratch_types=[
            pltpu.VMEM((CH,), jnp.int32),
            pltpu.VMEM((CH,), jnp.int32),
            pltpu.VMEM((CH,), jnp.int32),
            pltpu.VMEM((CH,), jnp.int32),
            pltpu.VMEM((CH,), jnp.int32),
            pltpu.VMEM((CH,), jnp.int32),
            pltpu.VMEM((CH,), jnp.float32),
            pltpu.VMEM((CH,), jnp.float32),
            pltpu.VMEM((RW,), jnp.float32),
            pltpu.VMEM((ZB,), jnp.float32),
            pltpu.VMEM((16,), jnp.int32),
            pltpu.VMEM((NROW, RW), jnp.int32),
            pltpu.VMEM((NROW, RW), jnp.float32),
            pltpu.VMEM_SHARED((ACC,), jnp.float32),
            pltpu.VMEM_SHARED((ACC,), jnp.float32),
            pltpu.SemaphoreType.DMA,
            pltpu.SemaphoreType.DMA,
        ],
    )(x, y, z, feat)


def _combine_body(f_ref, w_ref, fv_ref, cv_ref, of_ref, oc_ref):
    f = f_ref[...]
    w = w_ref[...]
    fv = fv_ref[...]
    cv = cv_ref[...]
    touched = w > 0.0
    pooled = f / w
    cv1 = cv + 1.0
    of_ref[...] = jnp.where(touched, fv * cv + pooled / cv1, fv)
    oc_ref[...] = jnp.where(touched, cv1, cv)


def _combine(fcache, wcache, fv, cv):
    n = 65536
    blk = 2048
    spec = pl.BlockSpec((blk, 256), lambda i: (i, 0))
    return pl.pallas_call(
        _combine_body,
        grid=(n // blk,),
        in_specs=[spec, spec, spec, spec],
        out_specs=[spec, spec],
        out_shape=[
            jax.ShapeDtypeStruct((n, 256), jnp.float32),
            jax.ShapeDtypeStruct((n, 256), jnp.float32),
        ],
    )(fcache, wcache, fv, cv)


def kernel(feature, indices, feature_volume, count_volume):
    xs, ys, zs = feature_volume.shape
    feat = feature.reshape(NPTS)
    idx = indices.reshape(NPTS, 3)
    x = idx[:, 0]
    y = idx[:, 1]
    z = idx[:, 2]
    fcache, wcache = _sc_scatter(x, y, z, feat)
    of, oc = _combine(
        fcache.reshape(65536, 256),
        wcache.reshape(65536, 256),
        feature_volume.reshape(65536, 256),
        count_volume.reshape(65536, 256),
    )
    return of.reshape(xs, ys, zs), oc.reshape(xs, ys, zs)


# speedup vs baseline: 22.7157x; 1.0702x over previous
import functools

import jax
import jax.numpy as jnp
from jax import lax
from jax.experimental import pallas as pl
from jax.experimental.pallas import tpu as pltpu
from jax.experimental.pallas import tpu_sc as plsc

NPTS = 8 * 512 * 512
VOL = 256 * 256 * 256
NSUB = 16
TPP = NPTS // NSUB
CH = 2048
NCHUNK = TPP // CH
SLICE = 256 * 256
XS_PER_ROUND = 10
HALF_X = 128
ROUND_WIDTHS = [XS_PER_ROUND] * (HALF_X // XS_PER_ROUND) + (
    [HALF_X % XS_PER_ROUND] if HALF_X % XS_PER_ROUND else [])
NV_MAX = XS_PER_ROUND * SLICE
DUMP_SZ = 4096
ACC = NV_MAX + DUMP_SZ
ZB = 8192
NROW = 64
RW = 128
RING = NROW * RW


def _sc_body(x_hbm, y_hbm, z_hbm, feat_hbm, f_out, w_out, flat_hbm,
             xb0, xb1, yb0, yb1, zb0, zb1, fb0, fb1, lb0, lb1,
             ones_row, zeros, wcur, irows, frows, facc, wacc, sem0, sem1):
    c = lax.axis_index("c")
    s = lax.axis_index("s")
    tile_base = s * TPP
    xb = [xb0, xb1]
    yb = [yb0, yb1]
    zb = [zb0, zb1]
    fb = [fb0, fb1]
    lb = [lb0, lb1]
    sems = [sem0, sem1]
    iota = lax.iota(jnp.int32, 16)

    def init(i, _):
        zeros[pl.ds(i * 16, 16)] = jnp.zeros((16,), jnp.float32)
        @pl.when(i < RW // 16)
        def _():
            ones_row[pl.ds(i * 16, 16)] = jnp.ones((16,), jnp.float32)
        return 0
    lax.fori_loop(0, ZB // 16, init, 0)
    wcur[pl.ds(0, 16)] = jnp.zeros((16,), jnp.int32)

    def issue(k, b):
        p0 = tile_base + k * CH
        pltpu.async_copy(x_hbm.at[pl.ds(p0, CH)], xb[b], sems[b])
        pltpu.async_copy(y_hbm.at[pl.ds(p0, CH)], yb[b], sems[b])
        pltpu.async_copy(z_hbm.at[pl.ds(p0, CH)], zb[b], sems[b])
        pltpu.async_copy(feat_hbm.at[pl.ds(p0, CH)], fb[b], sems[b])

    def drain(k, b):
        p0 = tile_base + k * CH
        pltpu.make_async_copy(x_hbm.at[pl.ds(p0, CH)], xb[b], sems[b]).wait()
        pltpu.make_async_copy(y_hbm.at[pl.ds(p0, CH)], yb[b], sems[b]).wait()
        pltpu.make_async_copy(z_hbm.at[pl.ds(p0, CH)], zb[b], sems[b]).wait()
        pltpu.make_async_copy(feat_hbm.at[pl.ds(p0, CH)], fb[b], sems[b]).wait()

    def flush_rows(w_scalar, f_scalar):
        nfull = (w_scalar >> 7) - f_scalar

        def fl(j, _):
            j2 = (f_scalar + j) & (NROW - 1)
            pltpu.sync_copy(frows.at[j2], facc.at[irows.at[j2]], add=True)
            pltpu.sync_copy(ones_row, wacc.at[irows.at[j2]], add=True)
            return 0
        lax.fori_loop(0, nfull, fl, 0)
        return f_scalar + nfull


    flat_sc = flat_hbm.at[c]

    def pre(k, bb):
        drain(k, bb)

        @pl.when(k + 1 < NCHUNK)
        def _():
            issue(k + 1, 1 - bb)

        def grp(g, _):
            for j in range(8):
                sl = pl.ds((g * 8 + j) * 16, 16)
                lb[bb][sl] = (xb[bb][sl] << 16) | (yb[bb][sl] << 8) | zb[bb][sl]
            return 0
        lax.fori_loop(0, CH // 128, grp, 0)
        pltpu.sync_copy(lb[bb], flat_sc.at[pl.ds(tile_base + k * CH, CH)])

    issue(0, 0)

    def pre2(i, _):
        pre(2 * i, 0)
        pre(2 * i + 1, 1)
        return 0
    lax.fori_loop(0, NCHUNK // 2, pre2, 0)

    def issue2(k, b):
        p0 = tile_base + k * CH
        pltpu.async_copy(flat_sc.at[pl.ds(p0, CH)], lb[b], sems[b])
        pltpu.async_copy(feat_hbm.at[pl.ds(p0, CH)], fb[b], sems[b])

    def drain2(k, b):
        p0 = tile_base + k * CH
        pltpu.make_async_copy(flat_sc.at[pl.ds(p0, CH)], lb[b], sems[b]).wait()
        pltpu.make_async_copy(feat_hbm.at[pl.ds(p0, CH)], fb[b], sems[b]).wait()

    fcur = jnp.int32(0)
    for r, width in enumerate(ROUND_WIDTHS):
        nv = width * SLICE
        base = (c * HALF_X + r * XS_PER_ROUND) * SLICE
        per_tile = nv // NSUB

        full, rem = per_tile // ZB, per_tile % ZB
        for j in range(full):
            off = s * per_tile + j * ZB
            pltpu.sync_copy(zeros, facc.at[pl.ds(off, ZB)])
            pltpu.sync_copy(zeros, wacc.at[pl.ds(off, ZB)])
        if rem:
            off = s * per_tile + full * ZB
            pltpu.sync_copy(zeros.at[pl.ds(0, rem)], facc.at[pl.ds(off, rem)])
            pltpu.sync_copy(zeros.at[pl.ds(0, rem)], wacc.at[pl.ds(off, rem)])
        plsc.subcore_barrier()

        issue2(0, 0)

        def proc(k, bb, f_):
            drain2(k, bb)

            @pl.when(k + 1 < NCHUNK)
            def _():
                issue2(k + 1, 1 - bb)

            def group(g, _):
                ms, ts, fs, cs = [], [], [], []
                for j in range(8):
                    sl = pl.ds((g * 8 + j) * 16, 16)
                    t = lb[bb][sl] - base
                    m = (t >= 0) & (t < nv)
                    ms.append(m)
                    ts.append(t)
                    fs.append(fb[bb][sl])
                    cs.append(plsc.all_reduce_population_count(m))
                b = wcur[pl.ds(0, 16)]
                bases = []
                for j in range(8):
                    bases.append(b)
                    b = b + cs[j]
                wcur[pl.ds(0, 16)] = b
                for j in range(8):
                    dest = (bases[j] + jnp.cumsum(jnp.where(ms[j], 1, 0))
                            - 1) & (RING - 1)
                    plsc.store_scatter(irows, [dest >> 7, dest & (RW - 1)],
                                       ts[j], mask=ms[j])
                    plsc.store_scatter(frows, [dest >> 7, dest & (RW - 1)],
                                       fs[j], mask=ms[j])
                return 0
            lax.fori_loop(0, CH // 128, group, 0)
            return flush_rows(jnp.max(wcur[pl.ds(0, 16)]), f_)

        def chunk2(i, f_):
            f_ = proc(2 * i, 0, f_)
            f_ = proc(2 * i + 1, 1, f_)
            return f_
        fcur = lax.fori_loop(0, NCHUNK // 2, chunk2, fcur)

        w_s = jnp.max(wcur[pl.ds(0, 16)])
        for k2 in range(8):
            p = (w_s + 16 * k2 + iota) & (RING - 1)
            plsc.store_scatter(irows, [p >> 7, p & (RW - 1)],
                               NV_MAX + (p & (RW - 1)))
        w_pad = (w_s & ~(RW - 1)) + RW
        fcur = flush_rows(w_pad, fcur)
        wcur[pl.ds(0, 16)] = (wcur[pl.ds(0, 16)] * 0) + w_pad
        plsc.subcore_barrier()

        dr = nv // NSUB
        hbase = base + s * dr
        pltpu.sync_copy(facc.at[pl.ds(s * dr, dr)], f_out.at[pl.ds(hbase, dr)])
        pltpu.sync_copy(wacc.at[pl.ds(s * dr, dr)], w_out.at[pl.ds(hbase, dr)])
        plsc.subcore_barrier()


def _sc_scatter(x, y, z, feat):
    mesh = plsc.VectorSubcoreMesh(core_axis_name="c", subcore_axis_name="s")
    return pl.kernel(
        _sc_body,
        out_type=[
            jax.ShapeDtypeStruct((VOL,), jnp.float32),
            jax.ShapeDtypeStruct((VOL,), jnp.float32),
            jax.ShapeDtypeStruct((2, NPTS), jnp.int32),
        ],
        mesh=mesh,
        compiler_params=pltpu.CompilerParams(needs_layout_passes=False),
        scratch_types=[
            pltpu.VMEM((CH,), jnp.int32),
            pltpu.VMEM((CH,), jnp.int32),
            pltpu.VMEM((CH,), jnp.int32),
            pltpu.VMEM((CH,), jnp.int32),
            pltpu.VMEM((CH,), jnp.int32),
            pltpu.VMEM((CH,), jnp.int32),
            pltpu.VMEM((CH,), jnp.float32),
            pltpu.VMEM((CH,), jnp.float32),
            pltpu.VMEM((CH,), jnp.int32),
            pltpu.VMEM((CH,), jnp.int32),
            pltpu.VMEM((RW,), jnp.float32),
            pltpu.VMEM((ZB,), jnp.float32),
            pltpu.VMEM((16,), jnp.int32),
            pltpu.VMEM((NROW, RW), jnp.int32),
            pltpu.VMEM((NROW, RW), jnp.float32),
            pltpu.VMEM_SHARED((ACC,), jnp.float32),
            pltpu.VMEM_SHARED((ACC,), jnp.float32),
            pltpu.SemaphoreType.DMA,
            pltpu.SemaphoreType.DMA,
        ],
    )(x, y, z, feat)


def _combine_body(f_ref, w_ref, fv_ref, cv_ref, of_ref, oc_ref):
    f = f_ref[...]
    w = w_ref[...]
    fv = fv_ref[...]
    cv = cv_ref[...]
    touched = w > 0.0
    pooled = f / w
    cv1 = cv + 1.0
    of_ref[...] = jnp.where(touched, fv * cv + pooled / cv1, fv)
    oc_ref[...] = jnp.where(touched, cv1, cv)


def _combine(fcache, wcache, fv, cv):
    n = 65536
    blk = 2048
    spec = pl.BlockSpec((blk, 256), lambda i: (i, 0))
    return pl.pallas_call(
        _combine_body,
        grid=(n // blk,),
        in_specs=[spec, spec, spec, spec],
        out_specs=[spec, spec],
        out_shape=[
            jax.ShapeDtypeStruct((n, 256), jnp.float32),
            jax.ShapeDtypeStruct((n, 256), jnp.float32),
        ],
    )(fcache, wcache, fv, cv)


def kernel(feature, indices, feature_volume, count_volume):
    xs, ys, zs = feature_volume.shape
    feat = feature.reshape(NPTS)
    idx = indices.reshape(NPTS, 3)
    x = idx[:, 0]
    y = idx[:, 1]
    z = idx[:, 2]
    fcache, wcache, _ = _sc_scatter(x, y, z, feat)
    of, oc = _combine(
        fcache.reshape(65536, 256),
        wcache.reshape(65536, 256),
        feature_volume.reshape(65536, 256),
        count_volume.reshape(65536, 256),
    )
    return of.reshape(xs, ys, zs), oc.reshape(xs, ys, zs)
